# Initial kernel scaffold; baseline (speedup 1.0000x reference)
import jax
import jax.numpy as jnp
from jax.experimental import pallas as pl


def kernel(H, pf_gid, port_nodes_flat, port_w_signed_flat, port_len):
    raise NotImplementedError("write your pallas kernel here")


# speedup vs baseline: 6.6763x; 6.6763x over previous
import functools
import math

import jax
import jax.numpy as jnp
from jax import lax
from jax.experimental import pallas as pl
from jax.experimental.pallas import tpu as pltpu
from jax.experimental.pallas import tpu_sc as plsc

NC, NS, LANES = 2, 16, 16
NW = NC * NS
K = 120
OCT = 8


def _worker_bounds(G, nw=NW, oct_sz=OCT):
    total = G * (G - 1) // 2
    b = [0]
    for w in range(1, nw):
        t = w * total / nw
        g = (1.0 + math.sqrt(1.0 + 8.0 * t)) / 2.0
        g = int(round(g / oct_sz)) * oct_sz
        g = max(b[-1], min(g, G))
        b.append(g)
    b.append(G)
    return b


def _sc_segment_sums(H, nodes_pad, w_pad, G):
    D = H.shape[1]
    R = D // LANES
    mesh = plsc.VectorSubcoreMesh(core_axis_name="c", subcore_axis_name="s")
    noct = G // OCT
    half = noct // 2
    assert noct == 4 * NW

    @functools.partial(
        pl.kernel,
        out_type=(
            jax.ShapeDtypeStruct((G, D), jnp.float32),
            jax.ShapeDtypeStruct((G, D), jnp.float32),
            jax.ShapeDtypeStruct((G, LANES), jnp.float32),
        ),
        mesh=mesh,
        scratch_types=[
            pltpu.VMEM((K + 8,), jnp.int32),
            pltpu.VMEM((K + 8,), jnp.float32),
            pltpu.VMEM((K + 8, 128), jnp.float32),
            pltpu.VMEM((OCT, 128), jnp.float32),
            pltpu.VMEM((OCT, 128), jnp.float32),
            pltpu.VMEM((OCT, LANES), jnp.float32),
            pltpu.SemaphoreType.DMA,
        ],
    )
    def seg_kernel(h_hbm, nodes_hbm, w_hbm, sa_hbm, ss_hbm, wl_hbm,
                   idx_v, w_v, rows_v, sta_v, sts_v, stw_v, sem):
        wid = lax.axis_index("s") * NC + lax.axis_index("c")
        lane = lax.iota(jnp.int32, LANES)
        gdnums = lax.GatherDimensionNumbers(
            offset_dims=(), collapsed_slice_dims=(0,), start_index_map=(0,))

        def bcast_lane(vec, jj):
            idx = jnp.full((LANES, 1), jj, jnp.int32)
            return lax.gather(
                vec, idx, gdnums, slice_sizes=(1,),
                mode=lax.GatherScatterMode.PROMISE_IN_BOUNDS)

        def slot_body(s, _):
            oc = jnp.where(
                s == 0, wid,
                jnp.where(s == 1, half - 1 - wid,
                          jnp.where(s == 2, half + wid, noct - 1 - wid)))
            g0 = oc * OCT
            for gg in range(OCT):
                g = g0 + gg
                seg_start = (g * (g - 1)) // 2
                seg_end = seg_start + g
                nchunks = (g + K - 1) // K

                def chunk_body(j, accs):
                    base = seg_start + j * K
                    base_al = pl.multiple_of(base - lax.rem(base, 8), 8)
                    lo = base - base_al
                    hi = jnp.minimum(seg_end, base + K) - base_al
                    pltpu.sync_copy(nodes_hbm.at[pl.ds(base_al, K + 8)],
                                    idx_v)
                    pltpu.sync_copy(w_hbm.at[pl.ds(base_al, K + 8)], w_v)
                    pltpu.async_copy(h_hbm.at[idx_v], rows_v, sem).wait()

                    def blk_body(t, accs):
                        wacc, vs = accs
                        p0 = pl.multiple_of(t * LANES, 8)
                        pos = t * LANES + lane
                        w16 = w_v[pl.ds(p0, LANES)]
                        w16 = jnp.where((pos >= lo) & (pos < hi), w16, 0.0)
                        wacc = wacc + jnp.abs(w16)
                        vs = list(vs)
                        for jj in range(LANES):
                            wv = bcast_lane(w16, jj)
                            wav = jnp.abs(wv)
                            p = t * LANES + jj
                            for r in range(R):
                                row = rows_v[p, pl.ds(r * LANES, LANES)]
                                vs[r] = vs[r] + wav * row
                                vs[R + r] = vs[R + r] + wv * row
                        return (wacc, tuple(vs))

                    return lax.fori_loop(0, (K + 8) // LANES, blk_body, accs)

                zero = jnp.zeros((LANES,), jnp.float32)
                wacc, vs = lax.fori_loop(
                    0, nchunks, chunk_body,
                    (zero, tuple(zero for _ in range(2 * R))))
                stw_v[gg, :] = wacc
                for r in range(R):
                    sta_v[gg, r * LANES:(r + 1) * LANES] = vs[r]
                    sts_v[gg, r * LANES:(r + 1) * LANES] = vs[R + r]
            pltpu.sync_copy(sta_v, sa_hbm.at[pl.ds(g0, OCT)])
            pltpu.sync_copy(sts_v, ss_hbm.at[pl.ds(g0, OCT)])
            pltpu.sync_copy(stw_v, wl_hbm.at[pl.ds(g0, OCT)])
            return 0

        lax.fori_loop(0, 4, slot_body, 0)

    return seg_kernel(H, nodes_pad, w_pad)


def _tc_finish(S_abs, S_sgn, Wl, pf_gid):
    B = pf_gid.shape[0]
    G, D = S_abs.shape
    BB = 512
    nblk = B // BB
    gid3 = pf_gid.reshape(nblk, 1, BB)

    def body(gid_ref, sa_ref, ss_ref, wl_ref, va_ref, vs_ref):
        gid = gid_ref[0, 0, :]
        onehot = (lax.broadcasted_iota(jnp.int32, (BB, G), 1)
                  == gid[:, None]).astype(jnp.float32)
        pa = jnp.dot(onehot, sa_ref[...],
                     preferred_element_type=jnp.float32)
        ps = jnp.dot(onehot, ss_ref[...],
                     preferred_element_type=jnp.float32)
        pw = jnp.dot(onehot, wl_ref[...],
                     preferred_element_type=jnp.float32)
        denom = jnp.maximum(jnp.sum(pw, axis=1, keepdims=True), 1e-8)
        va = pa / denom
        vs = ps / denom
        na = jnp.sqrt(jnp.sum(va * va, axis=1, keepdims=True))
        ns = jnp.sqrt(jnp.sum(vs * vs, axis=1, keepdims=True))
        va_ref[...] = va / jnp.maximum(na, 1e-6)
        vs_ref[...] = vs / jnp.maximum(ns, 1e-6)

    return pl.pallas_call(
        body,
        grid=(nblk,),
        in_specs=[
            pl.BlockSpec((1, 1, BB), lambda i: (i, 0, 0)),
            pl.BlockSpec((G, D), lambda i: (0, 0)),
            pl.BlockSpec((G, D), lambda i: (0, 0)),
            pl.BlockSpec((G, LANES), lambda i: (0, 0)),
        ],
        out_specs=[
            pl.BlockSpec((BB, D), lambda i: (i, 0)),
            pl.BlockSpec((BB, D), lambda i: (i, 0)),
        ],
        out_shape=[
            jax.ShapeDtypeStruct((B, D), jnp.float32),
            jax.ShapeDtypeStruct((B, D), jnp.float32),
        ],
    )(gid3, S_abs, S_sgn, Wl)


@jax.jit
def kernel(H, pf_gid, port_nodes_flat, port_w_signed_flat, port_len):
    G = port_len.shape[0]
    L = port_nodes_flat.shape[0]
    pad = K + 8
    nodes_pad = jnp.concatenate(
        [port_nodes_flat, jnp.zeros((pad,), jnp.int32)])
    w_pad = jnp.concatenate(
        [port_w_signed_flat, jnp.zeros((pad,), jnp.float32)])
    S_abs, S_sgn, Wl = _sc_segment_sums(H, nodes_pad, w_pad, G)
    return _tc_finish(S_abs, S_sgn, Wl, pf_gid)


# speedup vs baseline: 7.0391x; 1.0543x over previous
import functools
import math

import jax
import jax.numpy as jnp
from jax import lax
from jax.experimental import pallas as pl
from jax.experimental.pallas import tpu as pltpu
from jax.experimental.pallas import tpu_sc as plsc

NC, NS, LANES = 2, 16, 16
NW = NC * NS
K = 120
OCT = 8


def _worker_bounds(G, nw=NW, oct_sz=OCT):
    total = G * (G - 1) // 2
    b = [0]
    for w in range(1, nw):
        t = w * total / nw
        g = (1.0 + math.sqrt(1.0 + 8.0 * t)) / 2.0
        g = int(round(g / oct_sz)) * oct_sz
        g = max(b[-1], min(g, G))
        b.append(g)
    b.append(G)
    return b


def _sc_segment_sums(H, nodes_pad, w_pad, G):
    D = H.shape[1]
    R = D // LANES
    mesh = plsc.VectorSubcoreMesh(core_axis_name="c", subcore_axis_name="s")
    noct = G // OCT
    half = noct // 2
    assert noct == 4 * NW

    ODMA = 8296
    OBUF = 8416

    @functools.partial(
        pl.kernel,
        out_type=(
            jax.ShapeDtypeStruct((G, D), jnp.float32),
            jax.ShapeDtypeStruct((G, D), jnp.float32),
            jax.ShapeDtypeStruct((G, LANES), jnp.float32),
        ),
        mesh=mesh,
        scratch_types=[
            pltpu.VMEM((OBUF,), jnp.int32),
            pltpu.VMEM((OBUF,), jnp.float32),
            pltpu.VMEM((K + 8, 128), jnp.float32),
            pltpu.VMEM((K + 8, 128), jnp.float32),
            pltpu.VMEM((OCT, 128), jnp.float32),
            pltpu.VMEM((OCT, 128), jnp.float32),
            pltpu.VMEM((OCT, LANES), jnp.float32),
            pltpu.SemaphoreType.DMA,
            pltpu.SemaphoreType.DMA,
            pltpu.SemaphoreType.DMA,
        ],
    )
    def seg_kernel(h_hbm, nodes_hbm, w_hbm, sa_hbm, ss_hbm, wl_hbm,
                   idx_v, w_v, rows_a, rows_b, sta_v, sts_v, stw_v,
                   sem_a, sem_b, sem_s):
        wid = lax.axis_index("s") * NC + lax.axis_index("c")
        lane = lax.iota(jnp.int32, LANES)
        gdnums = lax.GatherDimensionNumbers(
            offset_dims=(), collapsed_slice_dims=(0,), start_index_map=(0,))

        def bcast_lane(vec, jj):
            idx = jnp.full((LANES, 1), jj, jnp.int32)
            return lax.gather(
                vec, idx, gdnums, slice_sizes=(1,),
                mode=lax.GatherScatterMode.PROMISE_IN_BOUNDS)

        rows = (rows_a, rows_b)
        sems = (sem_a, sem_b)

        def issue(loff, c, buf):
            start = loff + c * K
            start_al = pl.multiple_of(start - lax.rem(start, 8), 8)
            pltpu.async_copy(h_hbm.at[idx_v.at[pl.ds(start_al, K + 8)]],
                             rows[buf], sems[buf])

        def process(loff, seg_len, c, buf, accs):
            start = loff + c * K
            start_al = pl.multiple_of(start - lax.rem(start, 8), 8)
            lo = start - start_al
            hi = jnp.minimum(seg_len - c * K, K) + lo
            rv = rows[buf]

            def blk_body(t, accs):
                wacc, vs = accs
                p0 = pl.multiple_of(start_al + t * LANES, 8)
                pos = t * LANES + lane
                w16 = w_v[pl.ds(p0, LANES)]
                w16 = jnp.where((pos >= lo) & (pos < hi), w16, 0.0)
                wacc = wacc + jnp.abs(w16)
                vs = list(vs)
                for jj in range(LANES):
                    wv = bcast_lane(w16, jj)
                    wav = jnp.abs(wv)
                    p = t * LANES + jj
                    for r in range(R):
                        row = rv[p, pl.ds(r * LANES, LANES)]
                        vs[r] = vs[r] + wav * row
                        vs[R + r] = vs[R + r] + wv * row
                return (wacc, tuple(vs))

            return lax.fori_loop(0, (K + 8) // LANES, blk_body, accs)

        def slot_body(s, _):
            oc = jnp.where(
                s == 0, wid,
                jnp.where(s == 1, half - 1 - wid,
                          jnp.where(s == 2, half + wid, noct - 1 - wid)))
            g0 = oc * OCT
            so = 32 * oc * oc - 4 * oc
            so_al = pl.multiple_of(so - lax.rem(so, 8), 8)
            pltpu.async_copy(nodes_hbm.at[pl.ds(so_al, ODMA)],
                             idx_v.at[pl.ds(0, ODMA)], sem_s)
            pltpu.async_copy(w_hbm.at[pl.ds(so_al, ODMA)],
                             w_v.at[pl.ds(0, ODMA)], sem_s).wait()
            pltpu.make_async_copy(nodes_hbm.at[pl.ds(so_al, ODMA)],
                                  idx_v.at[pl.ds(0, ODMA)], sem_s).wait()

            def group_body(gg, _):
                g = g0 + gg
                seg_start = (g * (g - 1)) // 2
                loff = seg_start - so_al
                nch = lax.div(g + K - 1, K)
                zero = jnp.zeros((LANES,), jnp.float32)
                accs = (zero, tuple(zero for _ in range(2 * R)))

                @pl.when(nch > 0)
                def _():
                    issue(loff, 0, 0)

                @pl.when(nch > 1)
                def _():
                    issue(loff, 1, 1)

                def pair_body(pr, accs):
                    c0 = 2 * pr
                    pltpu.make_async_copy(h_hbm.at[idx_v.at[:K + 8]],
                                          rows[0], sems[0]).wait()
                    accs = process(loff, g, c0, 0, accs)

                    @pl.when(c0 + 2 < nch)
                    def _():
                        issue(loff, c0 + 2, 0)

                    @pl.when(c0 + 1 < nch)
                    def _():
                        pltpu.make_async_copy(h_hbm.at[idx_v.at[:K + 8]],
                                              rows[1], sems[1]).wait()

                    accs = process(loff, jnp.where(c0 + 1 < nch, g, 0),
                                   c0 + 1, 1, accs)

                    @pl.when(c0 + 3 < nch)
                    def _():
                        issue(loff, c0 + 3, 1)

                    return accs

                wacc, vs = lax.fori_loop(0, lax.div(nch + 1, 2), pair_body,
                                         accs)
                stw_v[gg, :] = wacc
                for r in range(R):
                    sta_v[gg, r * LANES:(r + 1) * LANES] = vs[r]
                    sts_v[gg, r * LANES:(r + 1) * LANES] = vs[R + r]
                return 0

            lax.fori_loop(0, OCT, group_body, 0)
            pltpu.sync_copy(sta_v, sa_hbm.at[pl.ds(g0, OCT)])
            pltpu.sync_copy(sts_v, ss_hbm.at[pl.ds(g0, OCT)])
            pltpu.sync_copy(stw_v, wl_hbm.at[pl.ds(g0, OCT)])
            return 0

        pltpu.sync_copy(nodes_hbm.at[pl.ds(0, OBUF)], idx_v)
        pltpu.async_copy(h_hbm.at[idx_v.at[:K + 8]], rows_a, sem_a).wait()
        pltpu.async_copy(h_hbm.at[idx_v.at[:K + 8]], rows_b, sem_b).wait()

        lax.fori_loop(0, 4, slot_body, 0)

    return seg_kernel(H, nodes_pad, w_pad)


def _tc_finish(S_abs, S_sgn, Wl, pf_gid):
    B = pf_gid.shape[0]
    G, D = S_abs.shape
    BB = 512
    nblk = B // BB
    gid3 = pf_gid.reshape(nblk, 1, BB)

    def body(gid_ref, sa_ref, ss_ref, wl_ref, va_ref, vs_ref):
        gid = gid_ref[0, 0, :]
        onehot = (lax.broadcasted_iota(jnp.int32, (BB, G), 1)
                  == gid[:, None]).astype(jnp.float32)
        pa = jnp.dot(onehot, sa_ref[...],
                     preferred_element_type=jnp.float32)
        ps = jnp.dot(onehot, ss_ref[...],
                     preferred_element_type=jnp.float32)
        pw = jnp.dot(onehot, wl_ref[...],
                     preferred_element_type=jnp.float32)
        denom = jnp.maximum(jnp.sum(pw, axis=1, keepdims=True), 1e-8)
        va = pa / denom
        vs = ps / denom
        na = jnp.sqrt(jnp.sum(va * va, axis=1, keepdims=True))
        ns = jnp.sqrt(jnp.sum(vs * vs, axis=1, keepdims=True))
        va_ref[...] = va / jnp.maximum(na, 1e-6)
        vs_ref[...] = vs / jnp.maximum(ns, 1e-6)

    return pl.pallas_call(
        body,
        grid=(nblk,),
        in_specs=[
            pl.BlockSpec((1, 1, BB), lambda i: (i, 0, 0)),
            pl.BlockSpec((G, D), lambda i: (0, 0)),
            pl.BlockSpec((G, D), lambda i: (0, 0)),
            pl.BlockSpec((G, LANES), lambda i: (0, 0)),
        ],
        out_specs=[
            pl.BlockSpec((BB, D), lambda i: (i, 0)),
            pl.BlockSpec((BB, D), lambda i: (i, 0)),
        ],
        out_shape=[
            jax.ShapeDtypeStruct((B, D), jnp.float32),
            jax.ShapeDtypeStruct((B, D), jnp.float32),
        ],
    )(gid3, S_abs, S_sgn, Wl)


@jax.jit
def kernel(H, pf_gid, port_nodes_flat, port_w_signed_flat, port_len):
    G = port_len.shape[0]
    L = port_nodes_flat.shape[0]
    pad = 256
    nodes_pad = jnp.concatenate(
        [port_nodes_flat, jnp.zeros((pad,), jnp.int32)])
    w_pad = jnp.concatenate(
        [port_w_signed_flat, jnp.zeros((pad,), jnp.float32)])
    S_abs, S_sgn, Wl = _sc_segment_sums(H, nodes_pad, w_pad, G)
    return _tc_finish(S_abs, S_sgn, Wl, pf_gid)


# speedup vs baseline: 10.0609x; 1.4293x over previous
import functools
import math

import jax
import jax.numpy as jnp
from jax import lax
from jax.experimental import pallas as pl
from jax.experimental.pallas import tpu as pltpu
from jax.experimental.pallas import tpu_sc as plsc

NC, NS, LANES = 2, 16, 16
NW = NC * NS
K = 120
OCT = 8


def _worker_bounds(G, nw=NW, oct_sz=OCT):
    total = G * (G - 1) // 2
    b = [0]
    for w in range(1, nw):
        t = w * total / nw
        g = (1.0 + math.sqrt(1.0 + 8.0 * t)) / 2.0
        g = int(round(g / oct_sz)) * oct_sz
        g = max(b[-1], min(g, G))
        b.append(g)
    b.append(G)
    return b


def _sc_segment_sums(H, nodes_pad, w_pad, G):
    D = H.shape[1]
    R = D // LANES
    mesh = plsc.VectorSubcoreMesh(core_axis_name="c", subcore_axis_name="s")
    noct = G // OCT
    half = noct // 2
    assert noct == 4 * NW

    ODMA = 8296
    OBUF = 8416

    @functools.partial(
        pl.kernel,
        out_type=(
            jax.ShapeDtypeStruct((G, D), jnp.float32),
            jax.ShapeDtypeStruct((G, D), jnp.float32),
            jax.ShapeDtypeStruct((G, LANES), jnp.float32),
        ),
        mesh=mesh,
        scratch_types=[
            pltpu.VMEM((OBUF,), jnp.int32),
            pltpu.VMEM((OBUF,), jnp.float32),
            pltpu.VMEM((K + 8, 128), jnp.float32),
            pltpu.VMEM((K + 8, 128), jnp.float32),
            pltpu.VMEM((OCT, 128), jnp.float32),
            pltpu.VMEM((OCT, 128), jnp.float32),
            pltpu.VMEM((OCT, LANES), jnp.float32),
            pltpu.SemaphoreType.DMA,
            pltpu.SemaphoreType.DMA,
            pltpu.SemaphoreType.DMA,
        ],
    )
    def seg_kernel(h_hbm, nodes_hbm, w_hbm, sa_hbm, ss_hbm, wl_hbm,
                   idx_v, w_v, rows_a, rows_b, sta_v, sts_v, stw_v,
                   sem_a, sem_b, sem_s):
        wid = lax.axis_index("s") * NC + lax.axis_index("c")
        lane = lax.iota(jnp.int32, LANES)
        gdnums = lax.GatherDimensionNumbers(
            offset_dims=(), collapsed_slice_dims=(0,), start_index_map=(0,))

        def bcast_lane(vec, jj):
            idx = jnp.full((LANES, 1), jj, jnp.int32)
            return lax.gather(
                vec, idx, gdnums, slice_sizes=(1,),
                mode=lax.GatherScatterMode.PROMISE_IN_BOUNDS)

        rows = (rows_a, rows_b)
        sems = (sem_a, sem_b)

        def issue(loff, c, buf):
            start = loff + c * K
            start_al = pl.multiple_of(start - lax.rem(start, 8), 8)
            pltpu.async_copy(h_hbm.at[idx_v.at[pl.ds(start_al, K + 8)]],
                             rows[buf], sems[buf])

        def process(loff, seg_len, c, buf, accs):
            start = loff + c * K
            start_al = pl.multiple_of(start - lax.rem(start, 8), 8)
            lo = start - start_al
            hi = jnp.minimum(seg_len - c * K, K) + lo
            rv = rows[buf]
            RH = R // 2
            wacc, vs = accs
            vs = list(vs)
            for ph in range(2):

                def blk_body(t, pac):
                    wac = pac[0]
                    pa = list(pac[1:])
                    p0 = pl.multiple_of(start_al + t * LANES, 8)
                    pos = t * LANES + lane
                    w16 = w_v[pl.ds(p0, LANES)]
                    w16 = jnp.where((pos >= lo) & (pos < hi), w16, 0.0)
                    if ph == 0:
                        wac = wac + jnp.abs(w16)
                    for jj in range(LANES):
                        wv = bcast_lane(w16, jj)
                        wav = jnp.abs(wv)
                        p = t * LANES + jj
                        for r in range(RH):
                            row = rv[p, pl.ds((ph * RH + r) * LANES, LANES)]
                            pa[r] = pa[r] + wav * row
                            pa[RH + r] = pa[RH + r] + wv * row
                    return (wac,) + tuple(pa)

                init = ((wacc,)
                        + tuple(vs[ph * RH + r] for r in range(RH))
                        + tuple(vs[R + ph * RH + r] for r in range(RH)))
                out = lax.fori_loop(0, (K + 8) // LANES, blk_body, init)
                wacc = out[0]
                for r in range(RH):
                    vs[ph * RH + r] = out[1 + r]
                    vs[R + ph * RH + r] = out[1 + RH + r]
            return (wacc, tuple(vs))

        def slot_body(s, _):
            oc = jnp.where(
                s == 0, wid,
                jnp.where(s == 1, half - 1 - wid,
                          jnp.where(s == 2, half + wid, noct - 1 - wid)))
            g0 = oc * OCT
            so = 32 * oc * oc - 4 * oc
            so_al = pl.multiple_of(so - lax.rem(so, 8), 8)
            pltpu.async_copy(nodes_hbm.at[pl.ds(so_al, ODMA)],
                             idx_v.at[pl.ds(0, ODMA)], sem_s)
            pltpu.async_copy(w_hbm.at[pl.ds(so_al, ODMA)],
                             w_v.at[pl.ds(0, ODMA)], sem_s).wait()
            pltpu.make_async_copy(nodes_hbm.at[pl.ds(so_al, ODMA)],
                                  idx_v.at[pl.ds(0, ODMA)], sem_s).wait()

            def group_body(gg, _):
                g = g0 + gg
                seg_start = (g * (g - 1)) // 2
                loff = seg_start - so_al
                nch = lax.div(g + K - 1, K)
                zero = jnp.zeros((LANES,), jnp.float32)
                accs = (zero, tuple(zero for _ in range(2 * R)))

                @pl.when(nch > 0)
                def _():
                    issue(loff, 0, 0)

                @pl.when(nch > 1)
                def _():
                    issue(loff, 1, 1)

                def pair_body(pr, accs):
                    c0 = 2 * pr
                    pltpu.make_async_copy(h_hbm.at[idx_v.at[:K + 8]],
                                          rows[0], sems[0]).wait()
                    accs = process(loff, g, c0, 0, accs)

                    @pl.when(c0 + 2 < nch)
                    def _():
                        issue(loff, c0 + 2, 0)

                    @pl.when(c0 + 1 < nch)
                    def _():
                        pltpu.make_async_copy(h_hbm.at[idx_v.at[:K + 8]],
                                              rows[1], sems[1]).wait()

                    accs = process(loff, jnp.where(c0 + 1 < nch, g, 0),
                                   c0 + 1, 1, accs)

                    @pl.when(c0 + 3 < nch)
                    def _():
                        issue(loff, c0 + 3, 1)

                    return accs

                wacc, vs = lax.fori_loop(0, lax.div(nch + 1, 2), pair_body,
                                         accs)
                stw_v[gg, :] = wacc
                for r in range(R):
                    sta_v[gg, r * LANES:(r + 1) * LANES] = vs[r]
                    sts_v[gg, r * LANES:(r + 1) * LANES] = vs[R + r]
                return 0

            lax.fori_loop(0, OCT, group_body, 0)
            pltpu.sync_copy(sta_v, sa_hbm.at[pl.ds(g0, OCT)])
            pltpu.sync_copy(sts_v, ss_hbm.at[pl.ds(g0, OCT)])
            pltpu.sync_copy(stw_v, wl_hbm.at[pl.ds(g0, OCT)])
            return 0

        pltpu.sync_copy(nodes_hbm.at[pl.ds(0, OBUF)], idx_v)
        pltpu.async_copy(h_hbm.at[idx_v.at[:K + 8]], rows_a, sem_a).wait()
        pltpu.async_copy(h_hbm.at[idx_v.at[:K + 8]], rows_b, sem_b).wait()

        lax.fori_loop(0, 4, slot_body, 0)

    return seg_kernel(H, nodes_pad, w_pad)


def _tc_finish(S_abs, S_sgn, Wl, pf_gid):
    B = pf_gid.shape[0]
    G, D = S_abs.shape
    BB = 512
    nblk = B // BB
    gid3 = pf_gid.reshape(nblk, 1, BB)

    def body(gid_ref, sa_ref, ss_ref, wl_ref, va_ref, vs_ref):
        gid = gid_ref[0, 0, :]
        onehot = (lax.broadcasted_iota(jnp.int32, (BB, G), 1)
                  == gid[:, None]).astype(jnp.float32)
        pa = jnp.dot(onehot, sa_ref[...],
                     preferred_element_type=jnp.float32)
        ps = jnp.dot(onehot, ss_ref[...],
                     preferred_element_type=jnp.float32)
        pw = jnp.dot(onehot, wl_ref[...],
                     preferred_element_type=jnp.float32)
        denom = jnp.maximum(jnp.sum(pw, axis=1, keepdims=True), 1e-8)
        va = pa / denom
        vs = ps / denom
        na = jnp.sqrt(jnp.sum(va * va, axis=1, keepdims=True))
        ns = jnp.sqrt(jnp.sum(vs * vs, axis=1, keepdims=True))
        va_ref[...] = va / jnp.maximum(na, 1e-6)
        vs_ref[...] = vs / jnp.maximum(ns, 1e-6)

    return pl.pallas_call(
        body,
        grid=(nblk,),
        in_specs=[
            pl.BlockSpec((1, 1, BB), lambda i: (i, 0, 0)),
            pl.BlockSpec((G, D), lambda i: (0, 0)),
            pl.BlockSpec((G, D), lambda i: (0, 0)),
            pl.BlockSpec((G, LANES), lambda i: (0, 0)),
        ],
        out_specs=[
            pl.BlockSpec((BB, D), lambda i: (i, 0)),
            pl.BlockSpec((BB, D), lambda i: (i, 0)),
        ],
        out_shape=[
            jax.ShapeDtypeStruct((B, D), jnp.float32),
            jax.ShapeDtypeStruct((B, D), jnp.float32),
        ],
    )(gid3, S_abs, S_sgn, Wl)


@jax.jit
def kernel(H, pf_gid, port_nodes_flat, port_w_signed_flat, port_len):
    G = port_len.shape[0]
    L = port_nodes_flat.shape[0]
    pad = 256
    nodes_pad = jnp.concatenate(
        [port_nodes_flat, jnp.zeros((pad,), jnp.int32)])
    w_pad = jnp.concatenate(
        [port_w_signed_flat, jnp.zeros((pad,), jnp.float32)])
    S_abs, S_sgn, Wl = _sc_segment_sums(H, nodes_pad, w_pad, G)
    return _tc_finish(S_abs, S_sgn, Wl, pf_gid)


# speedup vs baseline: 21.2541x; 2.1125x over previous
import functools
import math

import jax
import jax.numpy as jnp
from jax import lax
from jax.experimental import pallas as pl
from jax.experimental.pallas import tpu as pltpu
from jax.experimental.pallas import tpu_sc as plsc

NC, NS, LANES = 2, 16, 16
NW = NC * NS
K = 120
OCT = 8


def _worker_bounds(G, nw=NW, oct_sz=OCT):
    total = G * (G - 1) // 2
    b = [0]
    for w in range(1, nw):
        t = w * total / nw
        g = (1.0 + math.sqrt(1.0 + 8.0 * t)) / 2.0
        g = int(round(g / oct_sz)) * oct_sz
        g = max(b[-1], min(g, G))
        b.append(g)
    b.append(G)
    return b


def _sc_segment_sums(H, nodes_pad, w_pad, G):
    D = H.shape[1]
    R = D // LANES
    mesh = plsc.VectorSubcoreMesh(core_axis_name="c", subcore_axis_name="s")
    noct = G // OCT
    half = noct // 2
    assert noct == 4 * NW

    ODMA = 8296
    OBUF = 8416

    @functools.partial(
        pl.kernel,
        out_type=(
            jax.ShapeDtypeStruct((G, D), jnp.float32),
            jax.ShapeDtypeStruct((G, D), jnp.float32),
            jax.ShapeDtypeStruct((G, LANES), jnp.float32),
        ),
        mesh=mesh,
        scratch_types=[
            pltpu.VMEM((OBUF,), jnp.int32),
            pltpu.VMEM((OBUF,), jnp.float32),
            pltpu.VMEM((K + 8, 128), jnp.float32),
            pltpu.VMEM((K + 8, 128), jnp.float32),
            pltpu.VMEM((OCT, 128), jnp.float32),
            pltpu.VMEM((OCT, 128), jnp.float32),
            pltpu.VMEM((OCT, LANES), jnp.float32),
            pltpu.SemaphoreType.DMA,
            pltpu.SemaphoreType.DMA,
            pltpu.SemaphoreType.DMA,
        ],
    )
    def seg_kernel(h_hbm, nodes_hbm, w_hbm, sa_hbm, ss_hbm, wl_hbm,
                   idx_v, w_v, rows_a, rows_b, sta_v, sts_v, stw_v,
                   sem_a, sem_b, sem_s):
        wid = lax.axis_index("s") * NC + lax.axis_index("c")
        lane = lax.iota(jnp.int32, LANES)
        gdnums = lax.GatherDimensionNumbers(
            offset_dims=(), collapsed_slice_dims=(0,), start_index_map=(0,))

        def bcast_lane(vec, jj):
            idx = jnp.full((LANES, 1), jj, jnp.int32)
            return lax.gather(
                vec, idx, gdnums, slice_sizes=(1,),
                mode=lax.GatherScatterMode.PROMISE_IN_BOUNDS)

        rows = (rows_a, rows_b)
        sems = (sem_a, sem_b)

        def issue(loff, c, buf):
            start = loff + c * K
            start_al = pl.multiple_of(start - lax.rem(start, 8), 8)
            pltpu.async_copy(h_hbm.at[idx_v.at[pl.ds(start_al, K + 8)]],
                             rows[buf], sems[buf])

        def process(loff, seg_len, c, buf, accs):
            start = loff + c * K
            start_al = pl.multiple_of(start - lax.rem(start, 8), 8)
            lo = start - start_al
            hi = jnp.minimum(seg_len - c * K, K) + lo
            rv = rows[buf]
            RH = R // 2
            wacc, vs = accs
            vs = list(vs)
            for ph in range(2):

                def blk_body(t, pac):
                    wac = pac[0]
                    p0 = pl.multiple_of(start_al + t * LANES, 8)
                    pos = t * LANES + lane
                    w16 = w_v[pl.ds(p0, LANES)]
                    w16 = jnp.where((pos >= lo) & (pos < hi), w16, 0.0)
                    if ph == 0:
                        wac = wac + jnp.abs(w16)

                    def sub_body(q, pa):
                        pa = list(pa)
                        for jj in range(8):
                            wv = bcast_lane(w16, q * 8 + jj)
                            wav = jnp.abs(wv)
                            p = t * LANES + q * 8 + jj
                            for r in range(RH):
                                row = rv[p,
                                         pl.ds((ph * RH + r) * LANES, LANES)]
                                pa[r] = pa[r] + wav * row
                                pa[RH + r] = pa[RH + r] + wv * row
                        return tuple(pa)

                    pa = lax.fori_loop(0, 2, sub_body, tuple(pac[1:]))
                    return (wac,) + tuple(pa)

                init = ((wacc,)
                        + tuple(vs[ph * RH + r] for r in range(RH))
                        + tuple(vs[R + ph * RH + r] for r in range(RH)))
                out = lax.fori_loop(0, (K + 8) // LANES, blk_body, init)
                wacc = out[0]
                for r in range(RH):
                    vs[ph * RH + r] = out[1 + r]
                    vs[R + ph * RH + r] = out[1 + RH + r]
            return (wacc, tuple(vs))

        def slot_body(s, _):
            oc = jnp.where(
                s == 0, wid,
                jnp.where(s == 1, half - 1 - wid,
                          jnp.where(s == 2, half + wid, noct - 1 - wid)))
            g0 = oc * OCT
            so = 32 * oc * oc - 4 * oc
            so_al = pl.multiple_of(so - lax.rem(so, 8), 8)
            pltpu.async_copy(nodes_hbm.at[pl.ds(so_al, ODMA)],
                             idx_v.at[pl.ds(0, ODMA)], sem_s)
            pltpu.async_copy(w_hbm.at[pl.ds(so_al, ODMA)],
                             w_v.at[pl.ds(0, ODMA)], sem_s).wait()
            pltpu.make_async_copy(nodes_hbm.at[pl.ds(so_al, ODMA)],
                                  idx_v.at[pl.ds(0, ODMA)], sem_s).wait()

            def group_body(gg, _):
                g = g0 + gg
                seg_start = (g * (g - 1)) // 2
                loff = seg_start - so_al
                nch = lax.div(g + K - 1, K)
                zero = jnp.zeros((LANES,), jnp.float32)
                accs = (zero, tuple(zero for _ in range(2 * R)))

                @pl.when(nch > 0)
                def _():
                    issue(loff, 0, 0)

                @pl.when(nch > 1)
                def _():
                    issue(loff, 1, 1)

                def pair_body(pr, accs):
                    c0 = 2 * pr
                    pltpu.make_async_copy(h_hbm.at[idx_v.at[:K + 8]],
                                          rows[0], sems[0]).wait()
                    accs = process(loff, g, c0, 0, accs)

                    @pl.when(c0 + 2 < nch)
                    def _():
                        issue(loff, c0 + 2, 0)

                    @pl.when(c0 + 1 < nch)
                    def _():
                        pltpu.make_async_copy(h_hbm.at[idx_v.at[:K + 8]],
                                              rows[1], sems[1]).wait()

                    accs = process(loff, jnp.where(c0 + 1 < nch, g, 0),
                                   c0 + 1, 1, accs)

                    @pl.when(c0 + 3 < nch)
                    def _():
                        issue(loff, c0 + 3, 1)

                    return accs

                wacc, vs = lax.fori_loop(0, lax.div(nch + 1, 2), pair_body,
                                         accs)
                stw_v[gg, :] = wacc
                for r in range(R):
                    sta_v[gg, r * LANES:(r + 1) * LANES] = vs[r]
                    sts_v[gg, r * LANES:(r + 1) * LANES] = vs[R + r]
                return 0

            lax.fori_loop(0, OCT, group_body, 0)
            pltpu.sync_copy(sta_v, sa_hbm.at[pl.ds(g0, OCT)])
            pltpu.sync_copy(sts_v, ss_hbm.at[pl.ds(g0, OCT)])
            pltpu.sync_copy(stw_v, wl_hbm.at[pl.ds(g0, OCT)])
            return 0

        pltpu.sync_copy(nodes_hbm.at[pl.ds(0, OBUF)], idx_v)
        pltpu.async_copy(h_hbm.at[idx_v.at[:K + 8]], rows_a, sem_a).wait()
        pltpu.async_copy(h_hbm.at[idx_v.at[:K + 8]], rows_b, sem_b).wait()

        lax.fori_loop(0, 4, slot_body, 0)

    return seg_kernel(H, nodes_pad, w_pad)


def _tc_finish(S_abs, S_sgn, Wl, pf_gid):
    B = pf_gid.shape[0]
    G, D = S_abs.shape
    BB = 512
    nblk = B // BB
    gid3 = pf_gid.reshape(nblk, 1, BB)

    def body(gid_ref, sa_ref, ss_ref, wl_ref, va_ref, vs_ref):
        gid = gid_ref[0, 0, :]
        onehot = (lax.broadcasted_iota(jnp.int32, (BB, G), 1)
                  == gid[:, None]).astype(jnp.float32)
        pa = jnp.dot(onehot, sa_ref[...],
                     preferred_element_type=jnp.float32)
        ps = jnp.dot(onehot, ss_ref[...],
                     preferred_element_type=jnp.float32)
        pw = jnp.dot(onehot, wl_ref[...],
                     preferred_element_type=jnp.float32)
        denom = jnp.maximum(jnp.sum(pw, axis=1, keepdims=True), 1e-8)
        va = pa / denom
        vs = ps / denom
        na = jnp.sqrt(jnp.sum(va * va, axis=1, keepdims=True))
        ns = jnp.sqrt(jnp.sum(vs * vs, axis=1, keepdims=True))
        va_ref[...] = va / jnp.maximum(na, 1e-6)
        vs_ref[...] = vs / jnp.maximum(ns, 1e-6)

    return pl.pallas_call(
        body,
        grid=(nblk,),
        in_specs=[
            pl.BlockSpec((1, 1, BB), lambda i: (i, 0, 0)),
            pl.BlockSpec((G, D), lambda i: (0, 0)),
            pl.BlockSpec((G, D), lambda i: (0, 0)),
            pl.BlockSpec((G, LANES), lambda i: (0, 0)),
        ],
        out_specs=[
            pl.BlockSpec((BB, D), lambda i: (i, 0)),
            pl.BlockSpec((BB, D), lambda i: (i, 0)),
        ],
        out_shape=[
            jax.ShapeDtypeStruct((B, D), jnp.float32),
            jax.ShapeDtypeStruct((B, D), jnp.float32),
        ],
    )(gid3, S_abs, S_sgn, Wl)


@jax.jit
def kernel(H, pf_gid, port_nodes_flat, port_w_signed_flat, port_len):
    G = port_len.shape[0]
    L = port_nodes_flat.shape[0]
    pad = 256
    nodes_pad = jnp.concatenate(
        [port_nodes_flat, jnp.zeros((pad,), jnp.int32)])
    w_pad = jnp.concatenate(
        [port_w_signed_flat, jnp.zeros((pad,), jnp.float32)])
    S_abs, S_sgn, Wl = _sc_segment_sums(H, nodes_pad, w_pad, G)
    return _tc_finish(S_abs, S_sgn, Wl, pf_gid)


# speedup vs baseline: 25.5652x; 1.2028x over previous
import functools
import math

import jax
import jax.numpy as jnp
from jax import lax
from jax.experimental import pallas as pl
from jax.experimental.pallas import tpu as pltpu
from jax.experimental.pallas import tpu_sc as plsc

NC, NS, LANES = 2, 16, 16
NW = NC * NS
K = 120
OCT = 8


def _worker_bounds(G, nw=NW, oct_sz=OCT):
    total = G * (G - 1) // 2
    b = [0]
    for w in range(1, nw):
        t = w * total / nw
        g = (1.0 + math.sqrt(1.0 + 8.0 * t)) / 2.0
        g = int(round(g / oct_sz)) * oct_sz
        g = max(b[-1], min(g, G))
        b.append(g)
    b.append(G)
    return b


def _sc_segment_sums(H, nodes_pad, w_pad, G):
    D = H.shape[1]
    R = D // LANES
    mesh = plsc.VectorSubcoreMesh(core_axis_name="c", subcore_axis_name="s")
    noct = G // OCT
    half = noct // 2
    assert noct == 4 * NW

    ODMA = 8296
    OBUF = 8416

    @functools.partial(
        pl.kernel,
        out_type=(
            jax.ShapeDtypeStruct((G, D), jnp.float32),
            jax.ShapeDtypeStruct((G, D), jnp.float32),
            jax.ShapeDtypeStruct((G, LANES), jnp.float32),
        ),
        mesh=mesh,
        scratch_types=[
            pltpu.VMEM((OBUF,), jnp.int32),
            pltpu.VMEM((OBUF,), jnp.float32),
            pltpu.VMEM((K + 8, 128), jnp.float32),
            pltpu.VMEM((K + 8, 128), jnp.float32),
            pltpu.VMEM((OCT, 128), jnp.float32),
            pltpu.VMEM((OCT, 128), jnp.float32),
            pltpu.VMEM((OCT, LANES), jnp.float32),
            pltpu.SemaphoreType.DMA,
            pltpu.SemaphoreType.DMA,
            pltpu.SemaphoreType.DMA,
        ],
    )
    def seg_kernel(h_hbm, nodes_hbm, w_hbm, sa_hbm, ss_hbm, wl_hbm,
                   idx_v, w_v, rows_a, rows_b, sta_v, sts_v, stw_v,
                   sem_a, sem_b, sem_s):
        wid = lax.axis_index("s") * NC + lax.axis_index("c")
        lane = lax.iota(jnp.int32, LANES)
        gdnums = lax.GatherDimensionNumbers(
            offset_dims=(), collapsed_slice_dims=(0,), start_index_map=(0,))

        def bcast_lane(vec, jj):
            idx = jnp.full((LANES, 1), jj, jnp.int32)
            return lax.gather(
                vec, idx, gdnums, slice_sizes=(1,),
                mode=lax.GatherScatterMode.PROMISE_IN_BOUNDS)

        rows = (rows_a, rows_b)
        sems = (sem_a, sem_b)

        def issue(loff, c, buf):
            start = loff + c * K
            start_al = pl.multiple_of(start - lax.rem(start, 8), 8)
            pltpu.async_copy(h_hbm.at[idx_v.at[pl.ds(start_al, K + 8)]],
                             rows[buf], sems[buf])

        def process(loff, seg_len, c, buf, accs):
            start = loff + c * K
            start_al = pl.multiple_of(start - lax.rem(start, 8), 8)
            lo = start - start_al
            hi = jnp.minimum(seg_len - c * K, K) + lo
            nblk = lax.div(jnp.maximum(hi, 0) + (LANES - 1), LANES)
            rv = rows[buf]
            RH = R // 2
            wacc, vs = accs
            vs = list(vs)
            for ph in range(2):

                def blk_body(t, pac):
                    wac = pac[0]
                    p0 = pl.multiple_of(start_al + t * LANES, 8)
                    pos = t * LANES + lane
                    w16 = w_v[pl.ds(p0, LANES)]
                    w16 = jnp.where((pos >= lo) & (pos < hi), w16, 0.0)
                    if ph == 0:
                        wac = wac + jnp.abs(w16)

                    def sub_body(q, pa):
                        pa = list(pa)
                        for jj in range(8):
                            wv = bcast_lane(w16, q * 8 + jj)
                            wav = jnp.abs(wv)
                            p = t * LANES + q * 8 + jj
                            for r in range(RH):
                                row = rv[p,
                                         pl.ds((ph * RH + r) * LANES, LANES)]
                                pa[r] = pa[r] + wav * row
                                pa[RH + r] = pa[RH + r] + wv * row
                        return tuple(pa)

                    pa = lax.fori_loop(0, 2, sub_body, tuple(pac[1:]))
                    return (wac,) + tuple(pa)

                init = ((wacc,)
                        + tuple(vs[ph * RH + r] for r in range(RH))
                        + tuple(vs[R + ph * RH + r] for r in range(RH)))
                out = lax.fori_loop(0, nblk, blk_body, init)
                wacc = out[0]
                for r in range(RH):
                    vs[ph * RH + r] = out[1 + r]
                    vs[R + ph * RH + r] = out[1 + RH + r]
            return (wacc, tuple(vs))

        def slot_body(s, _):
            oc = jnp.where(
                s == 0, wid,
                jnp.where(s == 1, half - 1 - wid,
                          jnp.where(s == 2, half + wid, noct - 1 - wid)))
            g0 = oc * OCT
            so = 32 * oc * oc - 4 * oc
            so_al = pl.multiple_of(so - lax.rem(so, 8), 8)
            pltpu.async_copy(nodes_hbm.at[pl.ds(so_al, ODMA)],
                             idx_v.at[pl.ds(0, ODMA)], sem_s)
            pltpu.async_copy(w_hbm.at[pl.ds(so_al, ODMA)],
                             w_v.at[pl.ds(0, ODMA)], sem_s).wait()
            pltpu.make_async_copy(nodes_hbm.at[pl.ds(so_al, ODMA)],
                                  idx_v.at[pl.ds(0, ODMA)], sem_s).wait()

            def group_body(gg, _):
                g = g0 + gg
                seg_start = (g * (g - 1)) // 2
                loff = seg_start - so_al
                nch = lax.div(g + K - 1, K)
                zero = jnp.zeros((LANES,), jnp.float32)
                accs = (zero, tuple(zero for _ in range(2 * R)))

                @pl.when(nch > 0)
                def _():
                    issue(loff, 0, 0)

                @pl.when(nch > 1)
                def _():
                    issue(loff, 1, 1)

                def pair_body(pr, accs):
                    c0 = 2 * pr
                    pltpu.make_async_copy(h_hbm.at[idx_v.at[:K + 8]],
                                          rows[0], sems[0]).wait()
                    accs = process(loff, g, c0, 0, accs)

                    @pl.when(c0 + 2 < nch)
                    def _():
                        issue(loff, c0 + 2, 0)

                    @pl.when(c0 + 1 < nch)
                    def _():
                        pltpu.make_async_copy(h_hbm.at[idx_v.at[:K + 8]],
                                              rows[1], sems[1]).wait()

                    accs = process(loff, jnp.where(c0 + 1 < nch, g, 0),
                                   c0 + 1, 1, accs)

                    @pl.when(c0 + 3 < nch)
                    def _():
                        issue(loff, c0 + 3, 1)

                    return accs

                wacc, vs = lax.fori_loop(0, lax.div(nch + 1, 2), pair_body,
                                         accs)
                stw_v[gg, :] = wacc
                for r in range(R):
                    sta_v[gg, r * LANES:(r + 1) * LANES] = vs[r]
                    sts_v[gg, r * LANES:(r + 1) * LANES] = vs[R + r]
                return 0

            lax.fori_loop(0, OCT, group_body, 0)
            pltpu.sync_copy(sta_v, sa_hbm.at[pl.ds(g0, OCT)])
            pltpu.sync_copy(sts_v, ss_hbm.at[pl.ds(g0, OCT)])
            pltpu.sync_copy(stw_v, wl_hbm.at[pl.ds(g0, OCT)])
            return 0

        pltpu.sync_copy(nodes_hbm.at[pl.ds(0, OBUF)], idx_v)
        pltpu.async_copy(h_hbm.at[idx_v.at[:K + 8]], rows_a, sem_a).wait()
        pltpu.async_copy(h_hbm.at[idx_v.at[:K + 8]], rows_b, sem_b).wait()

        lax.fori_loop(0, 4, slot_body, 0)

    return seg_kernel(H, nodes_pad, w_pad)


def _tc_finish(S_abs, S_sgn, Wl, pf_gid):
    B = pf_gid.shape[0]
    G, D = S_abs.shape
    BB = 512
    nblk = B // BB
    gid3 = pf_gid.reshape(nblk, 1, BB)

    def body(gid_ref, sa_ref, ss_ref, wl_ref, va_ref, vs_ref):
        gid = gid_ref[0, 0, :]
        onehot = (lax.broadcasted_iota(jnp.int32, (BB, G), 1)
                  == gid[:, None]).astype(jnp.float32)
        pa = jnp.dot(onehot, sa_ref[...],
                     preferred_element_type=jnp.float32)
        ps = jnp.dot(onehot, ss_ref[...],
                     preferred_element_type=jnp.float32)
        pw = jnp.dot(onehot, wl_ref[...],
                     preferred_element_type=jnp.float32)
        denom = jnp.maximum(jnp.sum(pw, axis=1, keepdims=True), 1e-8)
        va = pa / denom
        vs = ps / denom
        na = jnp.sqrt(jnp.sum(va * va, axis=1, keepdims=True))
        ns = jnp.sqrt(jnp.sum(vs * vs, axis=1, keepdims=True))
        va_ref[...] = va / jnp.maximum(na, 1e-6)
        vs_ref[...] = vs / jnp.maximum(ns, 1e-6)

    return pl.pallas_call(
        body,
        grid=(nblk,),
        in_specs=[
            pl.BlockSpec((1, 1, BB), lambda i: (i, 0, 0)),
            pl.BlockSpec((G, D), lambda i: (0, 0)),
            pl.BlockSpec((G, D), lambda i: (0, 0)),
            pl.BlockSpec((G, LANES), lambda i: (0, 0)),
        ],
        out_specs=[
            pl.BlockSpec((BB, D), lambda i: (i, 0)),
            pl.BlockSpec((BB, D), lambda i: (i, 0)),
        ],
        out_shape=[
            jax.ShapeDtypeStruct((B, D), jnp.float32),
            jax.ShapeDtypeStruct((B, D), jnp.float32),
        ],
    )(gid3, S_abs, S_sgn, Wl)


@jax.jit
def kernel(H, pf_gid, port_nodes_flat, port_w_signed_flat, port_len):
    G = port_len.shape[0]
    L = port_nodes_flat.shape[0]
    pad = 256
    nodes_pad = jnp.concatenate(
        [port_nodes_flat, jnp.zeros((pad,), jnp.int32)])
    w_pad = jnp.concatenate(
        [port_w_signed_flat, jnp.zeros((pad,), jnp.float32)])
    S_abs, S_sgn, Wl = _sc_segment_sums(H, nodes_pad, w_pad, G)
    return _tc_finish(S_abs, S_sgn, Wl, pf_gid)


# speedup vs baseline: 25.5723x; 1.0003x over previous
import functools
import math

import jax
import jax.numpy as jnp
from jax import lax
from jax.experimental import pallas as pl
from jax.experimental.pallas import tpu as pltpu
from jax.experimental.pallas import tpu_sc as plsc

NC, NS, LANES = 2, 16, 16
NW = NC * NS
K = 120
OCT = 8


def _worker_bounds(G, nw=NW, oct_sz=OCT):
    total = G * (G - 1) // 2
    b = [0]
    for w in range(1, nw):
        t = w * total / nw
        g = (1.0 + math.sqrt(1.0 + 8.0 * t)) / 2.0
        g = int(round(g / oct_sz)) * oct_sz
        g = max(b[-1], min(g, G))
        b.append(g)
    b.append(G)
    return b


def _sc_segment_sums(H, nodes_pad, w_pad, G):
    D = H.shape[1]
    R = D // LANES
    mesh = plsc.VectorSubcoreMesh(core_axis_name="c", subcore_axis_name="s")
    noct = G // OCT
    half = noct // 2
    assert noct == 4 * NW

    ODMA = 8296
    OBUF = 8416

    @functools.partial(
        pl.kernel,
        out_type=(
            jax.ShapeDtypeStruct((G, D), jnp.float32),
            jax.ShapeDtypeStruct((G, D), jnp.float32),
            jax.ShapeDtypeStruct((G, LANES), jnp.float32),
        ),
        mesh=mesh,
        scratch_types=[
            pltpu.VMEM((OBUF,), jnp.int32),
            pltpu.VMEM((OBUF,), jnp.float32),
            pltpu.VMEM((K + 8, 128), jnp.float32),
            pltpu.VMEM((K + 8, 128), jnp.float32),
            pltpu.VMEM((OCT, 128), jnp.float32),
            pltpu.VMEM((OCT, 128), jnp.float32),
            pltpu.VMEM((OCT, LANES), jnp.float32),
            pltpu.SemaphoreType.DMA,
            pltpu.SemaphoreType.DMA,
            pltpu.SemaphoreType.DMA,
        ],
    )
    def seg_kernel(h_hbm, nodes_hbm, w_hbm, sa_hbm, ss_hbm, wl_hbm,
                   idx_v, w_v, rows_a, rows_b, sta_v, sts_v, stw_v,
                   sem_a, sem_b, sem_s):
        wid = lax.axis_index("s") * NC + lax.axis_index("c")
        lane = lax.iota(jnp.int32, LANES)
        gdnums = lax.GatherDimensionNumbers(
            offset_dims=(), collapsed_slice_dims=(0,), start_index_map=(0,))

        def bcast_lane(vec, jj):
            idx = jnp.full((LANES, 1), jj, jnp.int32)
            return lax.gather(
                vec, idx, gdnums, slice_sizes=(1,),
                mode=lax.GatherScatterMode.PROMISE_IN_BOUNDS)

        rows = (rows_a, rows_b)
        sems = (sem_a, sem_b)

        def issue(loff, c, buf):
            start = loff + c * K
            start_al = pl.multiple_of(start - lax.rem(start, 8), 8)
            hw = (K + 8) // 2
            pltpu.async_copy(h_hbm.at[idx_v.at[pl.ds(start_al, hw)]],
                             rows[buf].at[pl.ds(0, hw)], sems[buf])
            start2 = pl.multiple_of(start_al + hw, 8)
            pltpu.async_copy(h_hbm.at[idx_v.at[pl.ds(start2, hw)]],
                             rows[buf].at[pl.ds(hw, hw)], sems[buf])

        def process(loff, seg_len, c, buf, accs):
            start = loff + c * K
            start_al = pl.multiple_of(start - lax.rem(start, 8), 8)
            lo = start - start_al
            hi = jnp.minimum(seg_len - c * K, K) + lo
            nblk = lax.div(jnp.maximum(hi, 0) + (LANES - 1), LANES)
            rv = rows[buf]
            RH = R // 2
            wacc, vs = accs
            vs = list(vs)
            for ph in range(2):

                def blk_body(t, pac):
                    wac = pac[0]
                    p0 = pl.multiple_of(start_al + t * LANES, 8)
                    pos = t * LANES + lane
                    w16 = w_v[pl.ds(p0, LANES)]
                    w16 = jnp.where((pos >= lo) & (pos < hi), w16, 0.0)
                    if ph == 0:
                        wac = wac + jnp.abs(w16)

                    def sub_body(q, pa):
                        pa = list(pa)
                        for jj in range(8):
                            wv = bcast_lane(w16, q * 8 + jj)
                            wav = jnp.abs(wv)
                            p = t * LANES + q * 8 + jj
                            for r in range(RH):
                                row = rv[p,
                                         pl.ds((ph * RH + r) * LANES, LANES)]
                                pa[r] = pa[r] + wav * row
                                pa[RH + r] = pa[RH + r] + wv * row
                        return tuple(pa)

                    pa = lax.fori_loop(0, 2, sub_body, tuple(pac[1:]))
                    return (wac,) + tuple(pa)

                init = ((wacc,)
                        + tuple(vs[ph * RH + r] for r in range(RH))
                        + tuple(vs[R + ph * RH + r] for r in range(RH)))
                out = lax.fori_loop(0, nblk, blk_body, init)
                wacc = out[0]
                for r in range(RH):
                    vs[ph * RH + r] = out[1 + r]
                    vs[R + ph * RH + r] = out[1 + RH + r]
            return (wacc, tuple(vs))

        def slot_body(s, _):
            oc = jnp.where(
                s == 0, wid,
                jnp.where(s == 1, half - 1 - wid,
                          jnp.where(s == 2, half + wid, noct - 1 - wid)))
            g0 = oc * OCT
            so = 32 * oc * oc - 4 * oc
            so_al = pl.multiple_of(so - lax.rem(so, 8), 8)
            pltpu.async_copy(nodes_hbm.at[pl.ds(so_al, ODMA)],
                             idx_v.at[pl.ds(0, ODMA)], sem_s)
            pltpu.async_copy(w_hbm.at[pl.ds(so_al, ODMA)],
                             w_v.at[pl.ds(0, ODMA)], sem_s).wait()
            pltpu.make_async_copy(nodes_hbm.at[pl.ds(so_al, ODMA)],
                                  idx_v.at[pl.ds(0, ODMA)], sem_s).wait()

            def group_body(gg, _):
                g = g0 + gg
                seg_start = (g * (g - 1)) // 2
                loff = seg_start - so_al
                nch = lax.div(g + K - 1, K)
                zero = jnp.zeros((LANES,), jnp.float32)
                accs = (zero, tuple(zero for _ in range(2 * R)))

                @pl.when(nch > 0)
                def _():
                    issue(loff, 0, 0)

                @pl.when(nch > 1)
                def _():
                    issue(loff, 1, 1)

                def pair_body(pr, accs):
                    c0 = 2 * pr
                    pltpu.make_async_copy(h_hbm.at[idx_v.at[:K + 8]],
                                          rows[0], sems[0]).wait()
                    accs = process(loff, g, c0, 0, accs)

                    @pl.when(c0 + 2 < nch)
                    def _():
                        issue(loff, c0 + 2, 0)

                    @pl.when(c0 + 1 < nch)
                    def _():
                        pltpu.make_async_copy(h_hbm.at[idx_v.at[:K + 8]],
                                              rows[1], sems[1]).wait()

                    accs = process(loff, jnp.where(c0 + 1 < nch, g, 0),
                                   c0 + 1, 1, accs)

                    @pl.when(c0 + 3 < nch)
                    def _():
                        issue(loff, c0 + 3, 1)

                    return accs

                wacc, vs = lax.fori_loop(0, lax.div(nch + 1, 2), pair_body,
                                         accs)
                stw_v[gg, :] = wacc
                for r in range(R):
                    sta_v[gg, r * LANES:(r + 1) * LANES] = vs[r]
                    sts_v[gg, r * LANES:(r + 1) * LANES] = vs[R + r]
                return 0

            lax.fori_loop(0, OCT, group_body, 0)
            pltpu.sync_copy(sta_v, sa_hbm.at[pl.ds(g0, OCT)])
            pltpu.sync_copy(sts_v, ss_hbm.at[pl.ds(g0, OCT)])
            pltpu.sync_copy(stw_v, wl_hbm.at[pl.ds(g0, OCT)])
            return 0

        pltpu.sync_copy(nodes_hbm.at[pl.ds(0, OBUF)], idx_v)
        pltpu.async_copy(h_hbm.at[idx_v.at[:K + 8]], rows_a, sem_a).wait()
        pltpu.async_copy(h_hbm.at[idx_v.at[:K + 8]], rows_b, sem_b).wait()

        lax.fori_loop(0, 4, slot_body, 0)

    return seg_kernel(H, nodes_pad, w_pad)


def _tc_finish(S_abs, S_sgn, Wl, pf_gid):
    B = pf_gid.shape[0]
    G, D = S_abs.shape
    BB = 512
    nblk = B // BB
    gid3 = pf_gid.reshape(nblk, 1, BB)

    def body(gid_ref, sa_ref, ss_ref, wl_ref, va_ref, vs_ref):
        gid = gid_ref[0, 0, :]
        onehot = (lax.broadcasted_iota(jnp.int32, (BB, G), 1)
                  == gid[:, None]).astype(jnp.float32)
        pa = jnp.dot(onehot, sa_ref[...],
                     preferred_element_type=jnp.float32)
        ps = jnp.dot(onehot, ss_ref[...],
                     preferred_element_type=jnp.float32)
        pw = jnp.dot(onehot, wl_ref[...],
                     preferred_element_type=jnp.float32)
        denom = jnp.maximum(jnp.sum(pw, axis=1, keepdims=True), 1e-8)
        va = pa / denom
        vs = ps / denom
        na = jnp.sqrt(jnp.sum(va * va, axis=1, keepdims=True))
        ns = jnp.sqrt(jnp.sum(vs * vs, axis=1, keepdims=True))
        va_ref[...] = va / jnp.maximum(na, 1e-6)
        vs_ref[...] = vs / jnp.maximum(ns, 1e-6)

    return pl.pallas_call(
        body,
        grid=(nblk,),
        in_specs=[
            pl.BlockSpec((1, 1, BB), lambda i: (i, 0, 0)),
            pl.BlockSpec((G, D), lambda i: (0, 0)),
            pl.BlockSpec((G, D), lambda i: (0, 0)),
            pl.BlockSpec((G, LANES), lambda i: (0, 0)),
        ],
        out_specs=[
            pl.BlockSpec((BB, D), lambda i: (i, 0)),
            pl.BlockSpec((BB, D), lambda i: (i, 0)),
        ],
        out_shape=[
            jax.ShapeDtypeStruct((B, D), jnp.float32),
            jax.ShapeDtypeStruct((B, D), jnp.float32),
        ],
    )(gid3, S_abs, S_sgn, Wl)


@jax.jit
def kernel(H, pf_gid, port_nodes_flat, port_w_signed_flat, port_len):
    G = port_len.shape[0]
    L = port_nodes_flat.shape[0]
    pad = 256
    nodes_pad = jnp.concatenate(
        [port_nodes_flat, jnp.zeros((pad,), jnp.int32)])
    w_pad = jnp.concatenate(
        [port_w_signed_flat, jnp.zeros((pad,), jnp.float32)])
    S_abs, S_sgn, Wl = _sc_segment_sums(H, nodes_pad, w_pad, G)
    return _tc_finish(S_abs, S_sgn, Wl, pf_gid)


# speedup vs baseline: 26.2950x; 1.0283x over previous
import functools
import math

import jax
import jax.numpy as jnp
from jax import lax
from jax.experimental import pallas as pl
from jax.experimental.pallas import tpu as pltpu
from jax.experimental.pallas import tpu_sc as plsc

NC, NS, LANES = 2, 16, 16
NW = NC * NS
K = 120
OCT = 8


def _worker_bounds(G, nw=NW, oct_sz=OCT):
    total = G * (G - 1) // 2
    b = [0]
    for w in range(1, nw):
        t = w * total / nw
        g = (1.0 + math.sqrt(1.0 + 8.0 * t)) / 2.0
        g = int(round(g / oct_sz)) * oct_sz
        g = max(b[-1], min(g, G))
        b.append(g)
    b.append(G)
    return b


def _sc_segment_sums(H, nodes_pad, w_pad, G):
    D = H.shape[1]
    R = D // LANES
    mesh = plsc.VectorSubcoreMesh(core_axis_name="c", subcore_axis_name="s")
    noct = G // OCT
    half = noct // 2
    assert noct == 4 * NW

    ODMA = 8296
    OBUF = 8416

    @functools.partial(
        pl.kernel,
        out_type=(
            jax.ShapeDtypeStruct((G, D), jnp.float32),
            jax.ShapeDtypeStruct((G, D), jnp.float32),
            jax.ShapeDtypeStruct((G, LANES), jnp.float32),
        ),
        mesh=mesh,
        scratch_types=[
            pltpu.VMEM((OBUF,), jnp.int32),
            pltpu.VMEM((OBUF,), jnp.float32),
            pltpu.VMEM((K + 8, 128), jnp.float32),
            pltpu.VMEM((K + 8, 128), jnp.float32),
            pltpu.VMEM((OCT, 128), jnp.float32),
            pltpu.VMEM((OCT, 128), jnp.float32),
            pltpu.VMEM((OCT, LANES), jnp.float32),
            pltpu.SemaphoreType.DMA,
            pltpu.SemaphoreType.DMA,
            pltpu.SemaphoreType.DMA,
        ],
    )
    def seg_kernel(h_hbm, nodes_hbm, w_hbm, sa_hbm, ss_hbm, wl_hbm,
                   idx_v, w_v, rows_a, rows_b, sta_v, sts_v, stw_v,
                   sem_a, sem_b, sem_s):
        wid = lax.axis_index("s") * NC + lax.axis_index("c")
        lane = lax.iota(jnp.int32, LANES)
        gdnums = lax.GatherDimensionNumbers(
            offset_dims=(), collapsed_slice_dims=(0,), start_index_map=(0,))

        def bcast_lane(vec, jj):
            idx = jnp.full((LANES, 1), jj, jnp.int32)
            return lax.gather(
                vec, idx, gdnums, slice_sizes=(1,),
                mode=lax.GatherScatterMode.PROMISE_IN_BOUNDS)

        rows = (rows_a, rows_b)
        sems = (sem_a, sem_b)

        def issue(loff, c, buf):
            start = loff + c * K
            start_al = pl.multiple_of(start - lax.rem(start, 8), 8)
            hw = (K + 8) // 2
            pltpu.async_copy(h_hbm.at[idx_v.at[pl.ds(start_al, hw)]],
                             rows[buf].at[pl.ds(0, hw)], sems[buf])
            start2 = pl.multiple_of(start_al + hw, 8)
            pltpu.async_copy(h_hbm.at[idx_v.at[pl.ds(start2, hw)]],
                             rows[buf].at[pl.ds(hw, hw)], sems[buf])

        def process(loff, seg_len, c, buf, accs):
            start = loff + c * K
            start_al = pl.multiple_of(start - lax.rem(start, 8), 8)
            lo = start - start_al
            hi = jnp.minimum(seg_len - c * K, K) + lo
            nblk = lax.div(jnp.maximum(hi, 0) + (LANES - 1), LANES)
            rv = rows[buf]
            RH = R // 2
            wacc, vs = accs
            vs = list(vs)
            for ph in range(2):

                def blk_body(t, pac):
                    wac = pac[0]
                    p0 = pl.multiple_of(start_al + t * LANES, 8)
                    pos = t * LANES + lane
                    w16 = w_v[pl.ds(p0, LANES)]
                    w16 = jnp.where((pos >= lo) & (pos < hi), w16, 0.0)
                    if ph == 0:
                        wac = wac + jnp.abs(w16)

                    def sub_body(q, pa):
                        pa = list(pa)
                        ws = [bcast_lane(w16, q * 4 + jj) for jj in range(4)]
                        was = [jnp.abs(w) for w in ws]
                        for r in range(RH):
                            rs = [rv[t * LANES + q * 4 + jj,
                                     pl.ds((ph * RH + r) * LANES, LANES)]
                                  for jj in range(4)]
                            pa[r] = pa[r] + (
                                (was[0] * rs[0] + was[1] * rs[1])
                                + (was[2] * rs[2] + was[3] * rs[3]))
                            pa[RH + r] = pa[RH + r] + (
                                (ws[0] * rs[0] + ws[1] * rs[1])
                                + (ws[2] * rs[2] + ws[3] * rs[3]))
                        return tuple(pa)

                    pa = lax.fori_loop(0, 4, sub_body, tuple(pac[1:]))
                    return (wac,) + tuple(pa)

                init = ((wacc,)
                        + tuple(vs[ph * RH + r] for r in range(RH))
                        + tuple(vs[R + ph * RH + r] for r in range(RH)))
                out = lax.fori_loop(0, nblk, blk_body, init)
                wacc = out[0]
                for r in range(RH):
                    vs[ph * RH + r] = out[1 + r]
                    vs[R + ph * RH + r] = out[1 + RH + r]
            return (wacc, tuple(vs))

        def slot_body(s, _):
            oc = jnp.where(
                s == 0, wid,
                jnp.where(s == 1, half - 1 - wid,
                          jnp.where(s == 2, half + wid, noct - 1 - wid)))
            g0 = oc * OCT
            so = 32 * oc * oc - 4 * oc
            so_al = pl.multiple_of(so - lax.rem(so, 8), 8)
            pltpu.async_copy(nodes_hbm.at[pl.ds(so_al, ODMA)],
                             idx_v.at[pl.ds(0, ODMA)], sem_s)
            pltpu.async_copy(w_hbm.at[pl.ds(so_al, ODMA)],
                             w_v.at[pl.ds(0, ODMA)], sem_s).wait()
            pltpu.make_async_copy(nodes_hbm.at[pl.ds(so_al, ODMA)],
                                  idx_v.at[pl.ds(0, ODMA)], sem_s).wait()

            def group_body(gg, _):
                g = g0 + gg
                seg_start = (g * (g - 1)) // 2
                loff = seg_start - so_al
                nch = lax.div(g + K - 1, K)
                zero = jnp.zeros((LANES,), jnp.float32)
                accs = (zero, tuple(zero for _ in range(2 * R)))

                @pl.when(nch > 0)
                def _():
                    issue(loff, 0, 0)

                @pl.when(nch > 1)
                def _():
                    issue(loff, 1, 1)

                def pair_body(pr, accs):
                    c0 = 2 * pr
                    pltpu.make_async_copy(h_hbm.at[idx_v.at[:K + 8]],
                                          rows[0], sems[0]).wait()
                    accs = process(loff, g, c0, 0, accs)

                    @pl.when(c0 + 2 < nch)
                    def _():
                        issue(loff, c0 + 2, 0)

                    @pl.when(c0 + 1 < nch)
                    def _():
                        pltpu.make_async_copy(h_hbm.at[idx_v.at[:K + 8]],
                                              rows[1], sems[1]).wait()

                    accs = process(loff, jnp.where(c0 + 1 < nch, g, 0),
                                   c0 + 1, 1, accs)

                    @pl.when(c0 + 3 < nch)
                    def _():
                        issue(loff, c0 + 3, 1)

                    return accs

                wacc, vs = lax.fori_loop(0, lax.div(nch + 1, 2), pair_body,
                                         accs)
                stw_v[gg, :] = wacc
                for r in range(R):
                    sta_v[gg, r * LANES:(r + 1) * LANES] = vs[r]
                    sts_v[gg, r * LANES:(r + 1) * LANES] = vs[R + r]
                return 0

            lax.fori_loop(0, OCT, group_body, 0)
            pltpu.sync_copy(sta_v, sa_hbm.at[pl.ds(g0, OCT)])
            pltpu.sync_copy(sts_v, ss_hbm.at[pl.ds(g0, OCT)])
            pltpu.sync_copy(stw_v, wl_hbm.at[pl.ds(g0, OCT)])
            return 0

        pltpu.sync_copy(nodes_hbm.at[pl.ds(0, OBUF)], idx_v)
        pltpu.async_copy(h_hbm.at[idx_v.at[:K + 8]], rows_a, sem_a).wait()
        pltpu.async_copy(h_hbm.at[idx_v.at[:K + 8]], rows_b, sem_b).wait()

        lax.fori_loop(0, 4, slot_body, 0)

    return seg_kernel(H, nodes_pad, w_pad)


def _tc_finish(S_abs, S_sgn, Wl, pf_gid):
    B = pf_gid.shape[0]
    G, D = S_abs.shape
    BB = 512
    nblk = B // BB
    gid3 = pf_gid.reshape(nblk, 1, BB)

    def body(gid_ref, sa_ref, ss_ref, wl_ref, va_ref, vs_ref):
        gid = gid_ref[0, 0, :]
        onehot = (lax.broadcasted_iota(jnp.int32, (BB, G), 1)
                  == gid[:, None]).astype(jnp.float32)
        pa = jnp.dot(onehot, sa_ref[...],
                     preferred_element_type=jnp.float32)
        ps = jnp.dot(onehot, ss_ref[...],
                     preferred_element_type=jnp.float32)
        pw = jnp.dot(onehot, wl_ref[...],
                     preferred_element_type=jnp.float32)
        denom = jnp.maximum(jnp.sum(pw, axis=1, keepdims=True), 1e-8)
        va = pa / denom
        vs = ps / denom
        na = jnp.sqrt(jnp.sum(va * va, axis=1, keepdims=True))
        ns = jnp.sqrt(jnp.sum(vs * vs, axis=1, keepdims=True))
        va_ref[...] = va / jnp.maximum(na, 1e-6)
        vs_ref[...] = vs / jnp.maximum(ns, 1e-6)

    return pl.pallas_call(
        body,
        grid=(nblk,),
        in_specs=[
            pl.BlockSpec((1, 1, BB), lambda i: (i, 0, 0)),
            pl.BlockSpec((G, D), lambda i: (0, 0)),
            pl.BlockSpec((G, D), lambda i: (0, 0)),
            pl.BlockSpec((G, LANES), lambda i: (0, 0)),
        ],
        out_specs=[
            pl.BlockSpec((BB, D), lambda i: (i, 0)),
            pl.BlockSpec((BB, D), lambda i: (i, 0)),
        ],
        out_shape=[
            jax.ShapeDtypeStruct((B, D), jnp.float32),
            jax.ShapeDtypeStruct((B, D), jnp.float32),
        ],
    )(gid3, S_abs, S_sgn, Wl)


@jax.jit
def kernel(H, pf_gid, port_nodes_flat, port_w_signed_flat, port_len):
    G = port_len.shape[0]
    L = port_nodes_flat.shape[0]
    pad = 256
    nodes_pad = jnp.concatenate(
        [port_nodes_flat, jnp.zeros((pad,), jnp.int32)])
    w_pad = jnp.concatenate(
        [port_w_signed_flat, jnp.zeros((pad,), jnp.float32)])
    S_abs, S_sgn, Wl = _sc_segment_sums(H, nodes_pad, w_pad, G)
    return _tc_finish(S_abs, S_sgn, Wl, pf_gid)


# speedup vs baseline: 27.8905x; 1.0607x over previous
import functools
import math

import jax
import jax.numpy as jnp
from jax import lax
from jax.experimental import pallas as pl
from jax.experimental.pallas import tpu as pltpu
from jax.experimental.pallas import tpu_sc as plsc

NC, NS, LANES = 2, 16, 16
NW = NC * NS
K = 120
OCT = 8


def _worker_bounds(G, nw=NW, oct_sz=OCT):
    total = G * (G - 1) // 2
    b = [0]
    for w in range(1, nw):
        t = w * total / nw
        g = (1.0 + math.sqrt(1.0 + 8.0 * t)) / 2.0
        g = int(round(g / oct_sz)) * oct_sz
        g = max(b[-1], min(g, G))
        b.append(g)
    b.append(G)
    return b


def _sc_segment_sums(H, nodes_pad, w_pad, G):
    D = H.shape[1]
    R = D // LANES
    mesh = plsc.VectorSubcoreMesh(core_axis_name="c", subcore_axis_name="s")
    noct = G // OCT
    half = noct // 2
    assert noct == 4 * NW

    ODMA = 8296
    OBUF = 8416

    @functools.partial(
        pl.kernel,
        out_type=(
            jax.ShapeDtypeStruct((G, D), jnp.float32),
            jax.ShapeDtypeStruct((G, D), jnp.float32),
            jax.ShapeDtypeStruct((G, LANES), jnp.float32),
        ),
        mesh=mesh,
        scratch_types=[
            pltpu.VMEM((OBUF,), jnp.int32),
            pltpu.VMEM((OBUF,), jnp.float32),
            pltpu.VMEM((K + 8, 128), jnp.float32),
            pltpu.VMEM((K + 8, 128), jnp.float32),
            pltpu.VMEM((OCT, 128), jnp.float32),
            pltpu.VMEM((OCT, 128), jnp.float32),
            pltpu.VMEM((OCT, LANES), jnp.float32),
            pltpu.SemaphoreType.DMA,
            pltpu.SemaphoreType.DMA,
            pltpu.SemaphoreType.DMA,
        ],
    )
    def seg_kernel(h_hbm, nodes_hbm, w_hbm, sa_hbm, ss_hbm, wl_hbm,
                   idx_v, w_v, rows_a, rows_b, sta_v, sts_v, stw_v,
                   sem_a, sem_b, sem_s):
        wid = lax.axis_index("s") * NC + lax.axis_index("c")
        lane = lax.iota(jnp.int32, LANES)
        gdnums = lax.GatherDimensionNumbers(
            offset_dims=(), collapsed_slice_dims=(0,), start_index_map=(0,))

        def bcast_lane(vec, jj):
            idx = jnp.full((LANES, 1), jj, jnp.int32)
            return lax.gather(
                vec, idx, gdnums, slice_sizes=(1,),
                mode=lax.GatherScatterMode.PROMISE_IN_BOUNDS)

        rows = (rows_a, rows_b)
        sems = (sem_a, sem_b)

        def issue(loff, c, buf):
            start = loff + c * K
            start_al = pl.multiple_of(start - lax.rem(start, 8), 8)
            hw = (K + 8) // 2
            pltpu.async_copy(h_hbm.at[idx_v.at[pl.ds(start_al, hw)]],
                             rows[buf].at[pl.ds(0, hw)], sems[buf])
            start2 = pl.multiple_of(start_al + hw, 8)
            pltpu.async_copy(h_hbm.at[idx_v.at[pl.ds(start2, hw)]],
                             rows[buf].at[pl.ds(hw, hw)], sems[buf])

        def process(loff, seg_len, c, buf, accs):
            start = loff + c * K
            start_al = pl.multiple_of(start - lax.rem(start, 8), 8)
            lo = start - start_al
            hi = jnp.minimum(seg_len - c * K, K) + lo
            nblk = lax.div(jnp.maximum(hi, 0) + (LANES - 1), LANES)
            rv = rows[buf]
            RH = R // 2
            wacc, vs = accs
            vs = list(vs)
            for ph in range(2):

                def blk_body(t, pac):
                    wac = pac[0]
                    p0 = pl.multiple_of(start_al + t * LANES, 8)
                    pos = t * LANES + lane
                    w16 = w_v[pl.ds(p0, LANES)]
                    w16 = jnp.where((pos >= lo) & (pos < hi), w16, 0.0)
                    if ph == 0:
                        wac = wac + jnp.abs(w16)

                    def sub_body(q, pa):
                        pa = list(pa)
                        ws = [bcast_lane(w16, q * 4 + jj) for jj in range(4)]
                        was = [jnp.abs(w) for w in ws]
                        for r in range(RH):
                            rs = [rv[t * LANES + q * 4 + jj,
                                     pl.ds((ph * RH + r) * LANES, LANES)]
                                  for jj in range(4)]
                            pa[r] = pa[r] + (
                                (was[0] * rs[0] + was[1] * rs[1])
                                + (was[2] * rs[2] + was[3] * rs[3]))
                            pa[RH + r] = pa[RH + r] + (
                                (ws[0] * rs[0] + ws[1] * rs[1])
                                + (ws[2] * rs[2] + ws[3] * rs[3]))
                        return tuple(pa)

                    pa = lax.fori_loop(0, 4, sub_body, tuple(pac[1:]))
                    return (wac,) + tuple(pa)

                init = ((wacc,)
                        + tuple(vs[ph * RH + r] for r in range(RH))
                        + tuple(vs[R + ph * RH + r] for r in range(RH)))
                out = lax.fori_loop(0, nblk, blk_body, init)
                wacc = out[0]
                for r in range(RH):
                    vs[ph * RH + r] = out[1 + r]
                    vs[R + ph * RH + r] = out[1 + RH + r]
            return (wacc, tuple(vs))

        def slot_body(s, _):
            oc = jnp.where(
                s == 0, wid,
                jnp.where(s == 1, half - 1 - wid,
                          jnp.where(s == 2, half + wid, noct - 1 - wid)))
            g0 = oc * OCT
            so = 32 * oc * oc - 4 * oc
            so_al = pl.multiple_of(so - lax.rem(so, 8), 8)
            pltpu.async_copy(nodes_hbm.at[pl.ds(so_al, ODMA)],
                             idx_v.at[pl.ds(0, ODMA)], sem_s)
            pltpu.async_copy(w_hbm.at[pl.ds(so_al, ODMA)],
                             w_v.at[pl.ds(0, ODMA)], sem_s).wait()
            pltpu.make_async_copy(nodes_hbm.at[pl.ds(so_al, ODMA)],
                                  idx_v.at[pl.ds(0, ODMA)], sem_s).wait()

            def group_body(gg, _):
                g = g0 + gg
                seg_start = (g * (g - 1)) // 2
                loff = seg_start - so_al
                nch = lax.div(g + K - 1, K)
                prev_nch = jnp.where(gg > 0, lax.div(g - 1 + K - 1, K), 0)
                fresh = (gg == 0) | (prev_nch == 0)
                loff_n = loff + g
                zero = jnp.zeros((LANES,), jnp.float32)
                accs = (zero, tuple(zero for _ in range(2 * R)))

                @pl.when((nch > 0) & fresh)
                def _():
                    issue(loff, 0, 0)

                @pl.when((nch > 1) & (fresh | (prev_nch == 1)))
                def _():
                    issue(loff, 1, 1)

                def pair_body(pr, accs):
                    c0 = 2 * pr
                    pltpu.make_async_copy(h_hbm.at[idx_v.at[:K + 8]],
                                          rows[0], sems[0]).wait()
                    accs = process(loff, g, c0, 0, accs)

                    @pl.when(c0 + 2 < nch)
                    def _():
                        issue(loff, c0 + 2, 0)

                    @pl.when(((c0 + 2 == nch) | (c0 + 2 == nch + 1))
                             & (gg < OCT - 1))
                    def _():
                        issue(loff_n, 0, 0)

                    @pl.when(c0 + 1 < nch)
                    def _():
                        pltpu.make_async_copy(h_hbm.at[idx_v.at[:K + 8]],
                                              rows[1], sems[1]).wait()

                    accs = process(loff, jnp.where(c0 + 1 < nch, g, 0),
                                   c0 + 1, 1, accs)

                    @pl.when(c0 + 3 < nch)
                    def _():
                        issue(loff, c0 + 3, 1)

                    @pl.when(((c0 + 3 == nch) | (c0 + 3 == nch + 1))
                             & (gg < OCT - 1)
                             & (lax.div(g + 1 + K - 1, K) > 1))
                    def _():
                        issue(loff_n, 1, 1)

                    return accs

                wacc, vs = lax.fori_loop(0, lax.div(nch + 1, 2), pair_body,
                                         accs)
                stw_v[gg, :] = wacc
                for r in range(R):
                    sta_v[gg, r * LANES:(r + 1) * LANES] = vs[r]
                    sts_v[gg, r * LANES:(r + 1) * LANES] = vs[R + r]
                return 0

            lax.fori_loop(0, OCT, group_body, 0)
            pltpu.sync_copy(sta_v, sa_hbm.at[pl.ds(g0, OCT)])
            pltpu.sync_copy(sts_v, ss_hbm.at[pl.ds(g0, OCT)])
            pltpu.sync_copy(stw_v, wl_hbm.at[pl.ds(g0, OCT)])
            return 0

        pltpu.sync_copy(nodes_hbm.at[pl.ds(0, OBUF)], idx_v)
        pltpu.async_copy(h_hbm.at[idx_v.at[:K + 8]], rows_a, sem_a).wait()
        pltpu.async_copy(h_hbm.at[idx_v.at[:K + 8]], rows_b, sem_b).wait()

        lax.fori_loop(0, 4, slot_body, 0)

    return seg_kernel(H, nodes_pad, w_pad)


def _tc_finish(S_abs, S_sgn, Wl, pf_gid):
    B = pf_gid.shape[0]
    G, D = S_abs.shape
    BB = 512
    nblk = B // BB
    gid3 = pf_gid.reshape(nblk, 1, BB)

    def body(gid_ref, sa_ref, ss_ref, wl_ref, va_ref, vs_ref):
        gid = gid_ref[0, 0, :]
        onehot = (lax.broadcasted_iota(jnp.int32, (BB, G), 1)
                  == gid[:, None]).astype(jnp.float32)
        pa = jnp.dot(onehot, sa_ref[...],
                     preferred_element_type=jnp.float32)
        ps = jnp.dot(onehot, ss_ref[...],
                     preferred_element_type=jnp.float32)
        pw = jnp.dot(onehot, wl_ref[...],
                     preferred_element_type=jnp.float32)
        denom = jnp.maximum(jnp.sum(pw, axis=1, keepdims=True), 1e-8)
        va = pa / denom
        vs = ps / denom
        na = jnp.sqrt(jnp.sum(va * va, axis=1, keepdims=True))
        ns = jnp.sqrt(jnp.sum(vs * vs, axis=1, keepdims=True))
        va_ref[...] = va / jnp.maximum(na, 1e-6)
        vs_ref[...] = vs / jnp.maximum(ns, 1e-6)

    return pl.pallas_call(
        body,
        grid=(nblk,),
        in_specs=[
            pl.BlockSpec((1, 1, BB), lambda i: (i, 0, 0)),
            pl.BlockSpec((G, D), lambda i: (0, 0)),
            pl.BlockSpec((G, D), lambda i: (0, 0)),
            pl.BlockSpec((G, LANES), lambda i: (0, 0)),
        ],
        out_specs=[
            pl.BlockSpec((BB, D), lambda i: (i, 0)),
            pl.BlockSpec((BB, D), lambda i: (i, 0)),
        ],
        out_shape=[
            jax.ShapeDtypeStruct((B, D), jnp.float32),
            jax.ShapeDtypeStruct((B, D), jnp.float32),
        ],
    )(gid3, S_abs, S_sgn, Wl)


@jax.jit
def kernel(H, pf_gid, port_nodes_flat, port_w_signed_flat, port_len):
    G = port_len.shape[0]
    L = port_nodes_flat.shape[0]
    pad = 256
    nodes_pad = jnp.concatenate(
        [port_nodes_flat, jnp.zeros((pad,), jnp.int32)])
    w_pad = jnp.concatenate(
        [port_w_signed_flat, jnp.zeros((pad,), jnp.float32)])
    S_abs, S_sgn, Wl = _sc_segment_sums(H, nodes_pad, w_pad, G)
    return _tc_finish(S_abs, S_sgn, Wl, pf_gid)
